# Initial kernel scaffold; baseline (speedup 1.0000x reference)
import jax
import jax.numpy as jnp
from jax.experimental import pallas as pl


def kernel(features, edge_index, edge_weight, W1, b1, W2, b2, W3, b3, W4, b4, W5, b5, VW0, Vb0, VW1, Vb1, VW2, Vb2, VW3, Vb3, VW, Vb):
    raise NotImplementedError("write your pallas kernel here")


# speedup vs baseline: 3.0694x; 3.0694x over previous
import functools

import jax
import jax.numpy as jnp
from jax import lax
from jax.experimental import pallas as pl
from jax.experimental.pallas import tpu as pltpu
from jax.experimental.pallas import tpu_sc as plsc

N = 10000
NP = 10240
E = 320000
GAMMA = 0.1
K_ITERS = 6
D = 128

NCORES = 2
NTILES = 16
LANES = 16
PIECES = D // LANES
CHUNK = 128
CPT2 = -(-E // (32 * CHUNK))
EPT2 = CPT2 * CHUNK
CPT1 = 2 * CPT2
EPT1 = CPT1 * CHUNK
EP = EPT2 * 32
RPT = NP // NTILES
RSUB = 64


def _dense_body(act, has_skip, x_ref, wt_ref, b_ref, *rest):
    if has_skip:
        z_ref, o_ref = rest
    else:
        (o_ref,) = rest
    o = jax.lax.dot_general(
        x_ref[...], wt_ref[...], (((1,), (0,)), ((), ())),
        preferred_element_type=jnp.float32,
        precision=jax.lax.Precision.HIGHEST)
    o = o + b_ref[...]
    if has_skip:
        o = o + z_ref[...]
    if act == "elu":
        o = jnp.where(o > 0, o, jnp.exp(jnp.minimum(o, 0.0)) - 1.0)
    o_ref[...] = o


def _dense(x, W, b, z=None, act="none", block=1024):
    n, din = x.shape
    dout = W.shape[0]
    wt = W.T
    b2 = b.reshape(1, dout)
    in_specs = [
        pl.BlockSpec((block, din), lambda i: (i, 0)),
        pl.BlockSpec((din, dout), lambda i: (0, 0)),
        pl.BlockSpec((1, dout), lambda i: (0, 0)),
    ]
    args = [x, wt, b2]
    if z is not None:
        in_specs.append(pl.BlockSpec((block, dout), lambda i: (i, 0)))
        args.append(z)
    return pl.pallas_call(
        functools.partial(_dense_body, act, z is not None),
        grid=(n // block,),
        in_specs=in_specs,
        out_specs=pl.BlockSpec((block, dout), lambda i: (i, 0)),
        out_shape=jax.ShapeDtypeStruct((n, dout), jnp.float32),
    )(*args)


def _combine_body(p0_ref, p1_ref, xp_ref, o_ref):
    v = GAMMA * (p0_ref[...] + p1_ref[...]) + xp_ref[...]
    o_ref[...] = jnp.maximum(v, 0.0)


def _combine(p0, p1, xp, block=1024):
    spec = pl.BlockSpec((block, D), lambda i: (i, 0))
    return pl.pallas_call(
        _combine_body,
        grid=(NP // block,),
        in_specs=[spec, spec, spec],
        out_specs=spec,
        out_shape=jax.ShapeDtypeStruct((NP, D), jnp.float32),
    )(p0, p1, xp)


def _fill_zero(zero_v):
    @pl.loop(0, RSUB)
    def _(i):
        for p in range(PIECES):
            zero_v[i, pl.ds(p * LANES, LANES)] = jnp.zeros((LANES,), jnp.float32)


def _scale_rows(rows_v, wv_v):
    @pl.loop(0, CHUNK // LANES)
    def _(g):
        w16 = wv_v[pl.ds(g * LANES, LANES)]

        @pl.loop(0, LANES)
        def _(ii):
            wb = lax.gather(
                w16, jnp.full((LANES, 1), ii, jnp.int32),
                lax.GatherDimensionNumbers(
                    offset_dims=(), collapsed_slice_dims=(0,),
                    start_index_map=(0,)),
                (1,), mode=lax.GatherScatterMode.PROMISE_IN_BOUNDS)
            i = g * LANES + ii
            for p in range(PIECES):
                sl = (i, pl.ds(p * LANES, LANES))
                rows_v[sl] = rows_v[sl] * wb


def _edge_chunk(zsrc_hbm, src_hbm, dst_hbm, w_hbm, off, gbase,
                acc_sh, sidx_v, gidx_v, didx_v, rows_v, wv_v):
    pltpu.sync_copy(src_hbm.at[pl.ds(off, CHUNK)], sidx_v)
    pltpu.sync_copy(dst_hbm.at[pl.ds(off, CHUNK)], didx_v)
    pltpu.sync_copy(w_hbm.at[pl.ds(off, CHUNK)], wv_v)
    for q in range(CHUNK // LANES):
        sl = pl.ds(q * LANES, LANES)
        gidx_v[sl] = sidx_v[sl] + gbase
    pltpu.sync_copy(zsrc_hbm.at[gidx_v], rows_v)
    _scale_rows(rows_v, wv_v)
    pltpu.sync_copy(rows_v, acc_sh.at[didx_v], add=True)


_SCRATCH = [
    pltpu.VMEM_SHARED((NP, D), jnp.float32),
    pltpu.VMEM((CHUNK,), jnp.int32),
    pltpu.VMEM((CHUNK,), jnp.int32),
    pltpu.VMEM((CHUNK,), jnp.int32),
    pltpu.VMEM((CHUNK, D), jnp.float32),
    pltpu.VMEM((RSUB, D), jnp.float32),
    pltpu.VMEM((RSUB, D), jnp.float32),
    pltpu.VMEM((RSUB, D), jnp.float32),
    pltpu.VMEM((CHUNK,), jnp.float32),
]

_MESH = plsc.VectorSubcoreMesh(core_axis_name="core", subcore_axis_name="subcore")


def _ppr6_body(xp0_hbm, xp1_hbm, src_hbm, dst_hbm, w_hbm, z0_hbm, z1_hbm,
               acc_sh, sidx_v, gidx_v, didx_v, rows_v, abuf_v, xbuf_v, zero_v,
               wv_v):
    c = lax.axis_index("core")
    t = lax.axis_index("subcore")

    def run(xp_hbm, zbuf_hbm):
        _fill_zero(zero_v)
        for u in range(RPT // RSUB):
            r0 = t * RPT + u * RSUB
            pltpu.sync_copy(xp_hbm.at[pl.ds(r0, RSUB)], abuf_v)
            pltpu.sync_copy(abuf_v, zbuf_hbm.at[pl.ds(r0, RSUB)])
            pltpu.sync_copy(zero_v, acc_sh.at[pl.ds(r0, RSUB)])
        plsc.subcore_barrier()

        @pl.loop(0, K_ITERS)
        def _(k):
            gbase = k * NP

            @pl.loop(0, CPT1)
            def _(j):
                off = t * EPT1 + j * CHUNK
                _edge_chunk(zbuf_hbm, src_hbm, dst_hbm, w_hbm, off, gbase,
                            acc_sh, sidx_v, gidx_v, didx_v, rows_v, wv_v)

            plsc.subcore_barrier()

            wbase = (k + 1) * NP
            for u in range(RPT // RSUB):
                r0 = t * RPT + u * RSUB
                pltpu.sync_copy(acc_sh.at[pl.ds(r0, RSUB)], abuf_v)
                pltpu.sync_copy(zbuf_hbm.at[pl.ds(r0, RSUB)], xbuf_v)

                @pl.loop(0, RSUB)
                def _(i):
                    for p in range(PIECES):
                        sl = (i, pl.ds(p * LANES, LANES))
                        v = GAMMA * abuf_v[sl] + xbuf_v[sl]
                        abuf_v[sl] = jnp.maximum(v, 0.0)

                pltpu.sync_copy(abuf_v, zbuf_hbm.at[pl.ds(wbase + r0, RSUB)])
                pltpu.sync_copy(zero_v, acc_sh.at[pl.ds(r0, RSUB)])
            plsc.subcore_barrier()

    @pl.when(c == 0)
    def _():
        run(xp0_hbm, z0_hbm)

    @pl.when(c == 1)
    def _():
        run(xp1_hbm, z1_hbm)


def _ppr_layer1(xp, src_p, dst_p, w_p):
    zshape = jax.ShapeDtypeStruct(((K_ITERS + 1) * NP, D), jnp.float32)
    k = pl.kernel(_ppr6_body, out_type=(zshape, zshape), mesh=_MESH,
                  scratch_types=_SCRATCH)
    z0, z1 = k(xp[:, :D], xp[:, D:], src_p, dst_p, w_p)
    return jnp.concatenate([z0[K_ITERS * NP:], z1[K_ITERS * NP:]], axis=1)


def _spmm_body(z_hbm, src_hbm, dst_hbm, w_hbm, p0_hbm, p1_hbm,
               acc_sh, sidx_v, gidx_v, didx_v, rows_v, abuf_v, xbuf_v, zero_v,
               wv_v):
    c = lax.axis_index("core")
    t = lax.axis_index("subcore")

    _fill_zero(zero_v)
    for u in range(RPT // RSUB):
        r0 = t * RPT + u * RSUB
        pltpu.sync_copy(zero_v, acc_sh.at[pl.ds(r0, RSUB)])
    plsc.subcore_barrier()

    ebase = c * (EP // 2) + t * EPT2

    @pl.loop(0, CPT2)
    def _(j):
        _edge_chunk(z_hbm, src_hbm, dst_hbm, w_hbm, ebase + j * CHUNK, 0,
                    acc_sh, sidx_v, gidx_v, didx_v, rows_v, wv_v)

    plsc.subcore_barrier()

    def dump(p_hbm):
        for u in range(RPT // RSUB):
            r0 = t * RPT + u * RSUB
            pltpu.sync_copy(acc_sh.at[pl.ds(r0, RSUB)], abuf_v)
            pltpu.sync_copy(abuf_v, p_hbm.at[pl.ds(r0, RSUB)])

    @pl.when(c == 0)
    def _():
        dump(p0_hbm)

    @pl.when(c == 1)
    def _():
        dump(p1_hbm)


def _ppr_layer_iter(xp, src_p, dst_p, w_p):
    pshape = jax.ShapeDtypeStruct((NP, D), jnp.float32)
    spmm = pl.kernel(_spmm_body, out_type=(pshape, pshape), mesh=_MESH,
                     scratch_types=_SCRATCH)
    z = xp
    for _ in range(K_ITERS):
        p0, p1 = spmm(z, src_p, dst_p, w_p)
        z = _combine(p0, p1, xp)
    return z


def _pad_cols(a, d=D):
    return jnp.pad(a, ((0, 0), (0, d - a.shape[1])))


def kernel(features, edge_index, edge_weight,
           W1, b1, W2, b2, W3, b3, W4, b4, W5, b5,
           VW0, Vb0, VW1, Vb1, VW2, Vb2, VW3, Vb3, VW, Vb):
    pad = EP - E
    dst_p = jnp.concatenate([edge_index[0], jnp.zeros((pad,), jnp.int32)])
    src_p = jnp.concatenate([edge_index[1], jnp.zeros((pad,), jnp.int32)])
    w_p = jnp.concatenate([edge_weight, jnp.zeros((pad,), jnp.float32)])

    x = jnp.pad(features, ((0, NP - N), (0, 0)))
    z = _ppr_layer1(_dense(x, W1, b1), src_p, dst_p, w_p)
    x = _dense(x, VW0, Vb0, z=z, act="elu")
    z = _ppr_layer_iter(_dense(x, W2, b2), src_p, dst_p, w_p)
    x = _dense(x, VW1, Vb1, z=z, act="elu")
    z = _ppr_layer_iter(_dense(x, W3, b3), src_p, dst_p, w_p)
    x = _dense(x, VW2, Vb2, z=z, act="elu")
    W4p = jnp.pad(W4, ((0, 64), (0, 0)))
    b4p = jnp.pad(b4, (0, 64))
    z = _ppr_layer_iter(_dense(x, W4p, b4p), src_p, dst_p, w_p)[:, :64]
    x = _dense(x, VW3, Vb3, z=z, act="elu")
    W5p = jnp.pad(W5, ((0, 7), (0, 0)))
    b5p = jnp.pad(b5, (0, 7))
    VWp = jnp.pad(VW, ((0, 7), (0, 0)))
    Vbp = jnp.pad(Vb, (0, 7))
    z = _ppr_layer_iter(_dense(x, W5p, b5p), src_p, dst_p, w_p)
    out = _dense(x, VWp, Vbp, z=z)[:N, :121]
    return (out, K_ITERS * 5)
